# baseline (speedup 1.0000x reference)
import functools

import jax
import jax.numpy as jnp
from jax import lax
from jax.experimental import pallas as pl
from jax.experimental.pallas import tpu as pltpu
from jax.experimental.pallas import tpu_sc as plsc

MAX_LEN = 8192
D_MODEL = 768
NUM_CORES = 2
NUM_TILES = 16
NUM_WORKERS = NUM_CORES * NUM_TILES
ROWS_PER_WORKER = MAX_LEN // NUM_WORKERS
CHUNK = 64
NUM_CHUNKS = ROWS_PER_WORKER // CHUNK

_mesh = plsc.VectorSubcoreMesh(core_axis_name="c", subcore_axis_name="s")


@functools.partial(
    pl.kernel,
    mesh=_mesh,
    out_type=jax.ShapeDtypeStruct((MAX_LEN, D_MODEL), jnp.float32),
    scratch_types=(
        [pltpu.VMEM((NUM_CHUNKS, CHUNK), jnp.int32)]
        + [pltpu.SemaphoreType.DMA for _ in range(NUM_CHUNKS)]
    ),
)
def _gather_rows(emb_hbm, idx_hbm, out_hbm, idx_v, *sems):
    wid = lax.axis_index("s") * NUM_CORES + lax.axis_index("c")
    base = wid * ROWS_PER_WORKER
    pltpu.sync_copy(idx_hbm.at[wid], idx_v)
    copies = []
    for c in range(NUM_CHUNKS):
        copies.append(pltpu.async_copy(
            emb_hbm.at[pl.ds(base + c * CHUNK, CHUNK)],
            out_hbm.at[pl.ds(base + c * CHUNK, CHUNK)],
            sems[c]))
    for cp in copies:
        cp.wait()


def kernel(emb, n):
    n = jnp.asarray(n, jnp.int32)
    idx = jnp.minimum(jnp.arange(MAX_LEN, dtype=jnp.int32), n - 1)
    idx = idx.reshape(NUM_WORKERS, NUM_CHUNKS, CHUNK)
    return _gather_rows(emb, idx)


# speedup vs baseline: 21.1229x; 21.1229x over previous
import functools

import jax
import jax.numpy as jnp
from jax import lax
from jax.experimental import pallas as pl
from jax.experimental.pallas import tpu as pltpu
from jax.experimental.pallas import tpu_sc as plsc

MAX_LEN = 8192
D_MODEL = 768
NUM_CORES = 2
NUM_TILES = 16
NUM_WORKERS = NUM_CORES * NUM_TILES
ROWS_PER_WORKER = MAX_LEN // NUM_WORKERS
CHUNK = 32
NUM_CHUNKS = ROWS_PER_WORKER // CHUNK
NBUF = 5

_mesh = plsc.VectorSubcoreMesh(core_axis_name="c", subcore_axis_name="s")


@functools.partial(
    pl.kernel,
    mesh=_mesh,
    out_type=jax.ShapeDtypeStruct((MAX_LEN, D_MODEL), jnp.float32),
    scratch_types=(
        [pltpu.VMEM((NUM_CHUNKS, CHUNK), jnp.int32)]
        + [pltpu.VMEM((CHUNK, D_MODEL), jnp.float32) for _ in range(NBUF)]
        + [pltpu.SemaphoreType.DMA for _ in range(2 * NBUF)]
    ),
)
def _gather_rows(emb_hbm, idx_hbm, out_hbm, idx_v, *scratch):
    bufs = scratch[:NBUF]
    gsems = scratch[NBUF:2 * NBUF]
    ssems = scratch[2 * NBUF:]
    wid = lax.axis_index("s") * NUM_CORES + lax.axis_index("c")
    base = wid * ROWS_PER_WORKER
    pltpu.sync_copy(idx_hbm.at[wid], idx_v)

    gathers = [None] * NUM_CHUNKS
    scatters = [None] * NUM_CHUNKS

    for c in range(min(NBUF, NUM_CHUNKS)):
        gathers[c] = pltpu.async_copy(
            emb_hbm.at[idx_v.at[c]], bufs[c], gsems[c])
    for c in range(NUM_CHUNKS):
        b = c % NBUF
        prev = c - 1 + NBUF
        if 0 <= c - 1 and prev < NUM_CHUNKS:
            scatters[c - 1].wait()
            gathers[prev] = pltpu.async_copy(
                emb_hbm.at[idx_v.at[prev]], bufs[(c - 1) % NBUF],
                gsems[(c - 1) % NBUF])
        gathers[c].wait()
        scatters[c] = pltpu.async_copy(
            bufs[b], out_hbm.at[pl.ds(base + c * CHUNK, CHUNK)], ssems[b])
    for c in range(max(0, NUM_CHUNKS - NBUF), NUM_CHUNKS):
        scatters[c].wait()


def kernel(emb, n):
    n = jnp.asarray(n, jnp.int32)
    idx = jnp.minimum(jnp.arange(MAX_LEN, dtype=jnp.int32), n - 1)
    idx = idx.reshape(NUM_WORKERS, NUM_CHUNKS, CHUNK)
    return _gather_rows(emb, idx)
